# baseline (speedup 1.0000x reference)
import jax
import jax.numpy as jnp
from jax import lax
from jax.experimental import pallas as pl
from jax.experimental.pallas import tpu as pltpu
from jax.experimental.pallas import tpu_sc as plsc

D_MODEL = 512
MAX_REL = 32
SEQ_LEN = 512
NROWS = 2 * MAX_REL + 1
C = 8
EXTR = 88
LANES = 16
NV = D_MODEL // LANES
NB = 2 * SEQ_LEN

TC_ROWS = 256
SC_PER_W = (SEQ_LEN - TC_ROWS) // 32


def _sc_body(t_hbm, x_hbm, out_hbm, tbuf, xbuf, ext0, ext1, sem0, sem1):
    nc = 2
    wid = lax.axis_index("s") * nc + lax.axis_index("c")
    i0 = TC_ROWS + wid * SC_PER_W
    exts = (ext0, ext1)
    sems = (sem0, sem1)
    pltpu.sync_copy(t_hbm, tbuf)
    pltpu.sync_copy(x_hbm.at[pl.ds(i0, SC_PER_W)], xbuf)

    def build(t, ext):
        i = i0 + t
        q = C + lax.rem(i, C)
        xv = tuple(xbuf[t, pl.ds(v * LANES, LANES)] for v in range(NV))

        def build_row(m, xs):
            k = jnp.clip(m - q, 0, NROWS - 1)
            for v in range(NV):
                ext[m, pl.ds(v * LANES, LANES)] = tbuf[k, pl.ds(v * LANES, LANES)] + xs[v]
            return xs

        lax.fori_loop(0, EXTR, build_row, xv)

    def _cp(t, ext, sem, c2):
        i = i0 + t
        q = C + lax.rem(i, C)
        off = pl.multiple_of(jnp.clip(C * c2 - i + MAX_REL + q, 0, EXTR - C), C)
        dst = out_hbm.at[i, pl.ds(pl.multiple_of(C * c2, C), C)]
        return pltpu.make_async_copy(ext.at[pl.ds(off, C)], dst, sem)

    def fire(t, ext, sem):
        lax.fori_loop(0, SEQ_LEN // C, lambda c2, cc: (_cp(t, ext, sem, c2).start(), cc)[1], 0)

    def drain(t, ext, sem):
        lax.fori_loop(0, SEQ_LEN // C, lambda c2, cc: (_cp(t, ext, sem, c2).wait(), cc)[1], 0)

    def step(t, parity):
        ext = exts[parity]
        sem = sems[parity]
        drain(t - 2, ext, sem)
        build(t, ext)
        fire(t, ext, sem)

    build(0, ext0)
    fire(0, ext0, sem0)
    build(1, ext1)
    fire(1, ext1, sem1)

    def body_a(t, carry):
        step(t, 0)
        step(t + 1, 1)
        return carry

    lax.fori_loop(0, (SC_PER_W - 2) // 2, lambda s, cc: body_a(2 + 2 * s, cc), 0)

    drain(SC_PER_W - 2, ext0, sem0)
    drain(SC_PER_W - 1, ext1, sem1)


def _build_b_body(t_ref, b_ref):
    b_ref[0:480, :] = jnp.broadcast_to(t_ref[0:1, :], (480, D_MODEL))
    b_ref[480:544, :] = t_ref[1:65, :]
    b_ref[544:NB, :] = jnp.broadcast_to(t_ref[64:65, :], (NB - 544, D_MODEL))


def _tc_body(b_ref, x_ref, alias_ref, o_ref):
    del alias_ref
    g = pl.program_id(0)
    base = pl.multiple_of(8 * (SEQ_LEN // 8 - 2 - 2 * g), 8)
    v = b_ref[pl.ds(base, SEQ_LEN + 16), :]
    for u in range(16):
        o_ref[u] = v[15 - u : 15 - u + SEQ_LEN, :] + x_ref[u]


def kernel(x, relative_position_encoding):
    t = relative_position_encoding
    x2 = x.reshape(SEQ_LEN, D_MODEL)

    mesh = plsc.VectorSubcoreMesh(core_axis_name="c", subcore_axis_name="s")
    sc_out = pl.kernel(
        _sc_body,
        out_type=jax.ShapeDtypeStruct((SEQ_LEN, SEQ_LEN, D_MODEL), jnp.float32),
        mesh=mesh,
        scratch_types=[
            pltpu.VMEM((NROWS, D_MODEL), jnp.float32),
            pltpu.VMEM((SC_PER_W, D_MODEL), jnp.float32),
            pltpu.VMEM((EXTR, D_MODEL), jnp.float32),
            pltpu.VMEM((EXTR, D_MODEL), jnp.float32),
            pltpu.SemaphoreType.DMA,
            pltpu.SemaphoreType.DMA,
        ],
    )(t, x2)

    b = pl.pallas_call(
        _build_b_body,
        out_shape=jax.ShapeDtypeStruct((NB, D_MODEL), jnp.float32),
    )(t)

    out = pl.pallas_call(
        _tc_body,
        grid=(TC_ROWS // 16,),
        in_specs=[
            pl.BlockSpec((NB, D_MODEL), lambda g: (0, 0)),
            pl.BlockSpec((16, 1, D_MODEL), lambda g: (g, 0, 0)),
            pl.BlockSpec(memory_space=pltpu.MemorySpace.HBM),
        ],
        out_specs=pl.BlockSpec((16, SEQ_LEN, D_MODEL), lambda g: (g, 0, 0)),
        out_shape=jax.ShapeDtypeStruct((SEQ_LEN, SEQ_LEN, D_MODEL), jnp.float32),
        input_output_aliases={2: 0},
    )(b, x, sc_out)
    return out


# speedup vs baseline: 1.0073x; 1.0073x over previous
import jax
import jax.numpy as jnp
from jax import lax
from jax.experimental import pallas as pl
from jax.experimental.pallas import tpu as pltpu
from jax.experimental.pallas import tpu_sc as plsc

D_MODEL = 512
MAX_REL = 32
SEQ_LEN = 512
NROWS = 2 * MAX_REL + 1
C = 8
EXTR = 88
LANES = 16
NV = D_MODEL // LANES
NB = 2 * SEQ_LEN

TC_ROWS = 256
SC_PER_W = (SEQ_LEN - TC_ROWS) // 32


def _sc_body(t_hbm, x_hbm, out_hbm, tbuf, xbuf, ext0, ext1, sem0, sem1):
    nc = 2
    wid = lax.axis_index("s") * nc + lax.axis_index("c")
    i0 = TC_ROWS + wid * SC_PER_W
    exts = (ext0, ext1)
    sems = (sem0, sem1)
    pltpu.sync_copy(t_hbm, tbuf)
    pltpu.sync_copy(x_hbm.at[pl.ds(i0, SC_PER_W)], xbuf)

    def build(t, ext):
        i = i0 + t
        q = C + lax.rem(i, C)
        xv = tuple(xbuf[t, pl.ds(v * LANES, LANES)] for v in range(NV))

        def build_row(m, xs):
            k = jnp.clip(m - q, 0, NROWS - 1)
            for v in range(NV):
                ext[m, pl.ds(v * LANES, LANES)] = tbuf[k, pl.ds(v * LANES, LANES)] + xs[v]
            return xs

        lax.fori_loop(0, EXTR, build_row, xv)

    def _cp(t, ext, sem, c2):
        i = i0 + t
        q = C + lax.rem(i, C)
        off = pl.multiple_of(jnp.clip(C * c2 - i + MAX_REL + q, 0, EXTR - C), C)
        dst = out_hbm.at[i, pl.ds(pl.multiple_of(C * c2, C), C)]
        return pltpu.make_async_copy(ext.at[pl.ds(off, C)], dst, sem)

    def fire(t, ext, sem):
        lax.fori_loop(0, SEQ_LEN // C, lambda c2, cc: (_cp(t, ext, sem, c2).start(), cc)[1], 0)

    def drain(t, ext, sem):
        lax.fori_loop(0, SEQ_LEN // C, lambda c2, cc: (_cp(t, ext, sem, c2).wait(), cc)[1], 0)

    def step(t, parity):
        ext = exts[parity]
        sem = sems[parity]
        drain(t - 2, ext, sem)
        build(t, ext)
        fire(t, ext, sem)

    build(0, ext0)
    fire(0, ext0, sem0)
    build(1, ext1)
    fire(1, ext1, sem1)

    def body_a(t, carry):
        step(t, 0)
        step(t + 1, 1)
        return carry

    lax.fori_loop(0, (SC_PER_W - 2) // 2, lambda s, cc: body_a(2 + 2 * s, cc), 0)

    drain(SC_PER_W - 2, ext0, sem0)
    drain(SC_PER_W - 1, ext1, sem1)


def _build_b_body(t_ref, b_ref):
    b_ref[0:480, :] = jnp.broadcast_to(t_ref[0:1, :], (480, D_MODEL))
    b_ref[480:544, :] = t_ref[1:65, :]
    b_ref[544:NB, :] = jnp.broadcast_to(t_ref[64:65, :], (NB - 544, D_MODEL))


def _tc_body(b_ref, x_ref, alias_ref, o_ref):
    del alias_ref
    g = pl.program_id(0)
    base = pl.multiple_of(8 * (SEQ_LEN // 8 - 1 - g), 8)
    v = b_ref[pl.ds(base, SEQ_LEN + 8), :]
    for u in range(8):
        o_ref[u] = v[7 - u : 7 - u + SEQ_LEN, :] + x_ref[u]


def kernel(x, relative_position_encoding):
    t = relative_position_encoding
    x2 = x.reshape(SEQ_LEN, D_MODEL)

    mesh = plsc.VectorSubcoreMesh(core_axis_name="c", subcore_axis_name="s")
    sc_out = pl.kernel(
        _sc_body,
        out_type=jax.ShapeDtypeStruct((SEQ_LEN, SEQ_LEN, D_MODEL), jnp.float32),
        mesh=mesh,
        scratch_types=[
            pltpu.VMEM((NROWS, D_MODEL), jnp.float32),
            pltpu.VMEM((SC_PER_W, D_MODEL), jnp.float32),
            pltpu.VMEM((EXTR, D_MODEL), jnp.float32),
            pltpu.VMEM((EXTR, D_MODEL), jnp.float32),
            pltpu.SemaphoreType.DMA,
            pltpu.SemaphoreType.DMA,
        ],
    )(t, x2)

    b = pl.pallas_call(
        _build_b_body,
        out_shape=jax.ShapeDtypeStruct((NB, D_MODEL), jnp.float32),
    )(t)

    out = pl.pallas_call(
        _tc_body,
        grid=(TC_ROWS // 8,),
        in_specs=[
            pl.BlockSpec((NB, D_MODEL), lambda g: (0, 0)),
            pl.BlockSpec((8, 1, D_MODEL), lambda g: (g, 0, 0)),
            pl.BlockSpec(memory_space=pltpu.MemorySpace.HBM),
        ],
        out_specs=pl.BlockSpec((8, SEQ_LEN, D_MODEL), lambda g: (g, 0, 0)),
        out_shape=jax.ShapeDtypeStruct((SEQ_LEN, SEQ_LEN, D_MODEL), jnp.float32),
        input_output_aliases={2: 0},
    )(b, x, sc_out)
    return out
